# baseline (speedup 1.0000x reference)
import functools

import jax
import jax.numpy as jnp
from jax import lax
from jax.experimental import pallas as pl
from jax.experimental.pallas import tpu as pltpu
from jax.experimental.pallas import tpu_sc as plsc

INPUT_WIDTH = 128
OUTPUT_WIDTH = 64
N_CUTS = 20
BATCH = 16384
HALF = BATCH // 2

NUM_SC_CORES = 2
NUM_SC_SUBCORES = 16
NUM_WORKERS = NUM_SC_CORES * NUM_SC_SUBCORES

TC_BLOCK = 2048
N_TC_BLOCKS = HALF // TC_BLOCK
B_PER_W = HALF // NUM_WORKERS
N_CHUNKS = B_PER_W // 128
N_SEM_GROUPS = 4
C_PER_GROUP = OUTPUT_WIDTH // N_SEM_GROUPS
W_PER_BLOCK = TC_BLOCK // B_PER_W


def _tc_body(x_ref, cuts_ref, cw_ref, contrib_ref, idx_ref):
    xb = x_ref[...]
    cuts = cuts_ref[...]
    logits_t = lax.dot_general(
        cuts, xb, (((1,), (1,)), ((), ())),
        preferred_element_type=jnp.float32)
    act = jax.nn.sigmoid(logits_t)
    w0 = cw_ref[:, 0, :]
    w1 = cw_ref[:, 1, :]
    wdiff = w0 - w1
    wbase = jnp.sum(w1, axis=0)
    contrib_t = lax.dot_general(
        wdiff, act, (((0,), (0,)), ((), ())),
        precision=lax.Precision.HIGHEST,
        preferred_element_type=jnp.float32) + wbase[:, None]
    contrib_ref[...] = contrib_t[None]
    pow2 = (2 ** (N_CUTS - 1 - lax.broadcasted_iota(
        jnp.int32, (N_CUTS, 1), 0)))
    bits = jnp.where(logits_t > 0.0, pow2, 0)
    idx_ref[...] = jnp.sum(bits, axis=0, keepdims=True)[None]


def _tc_stage(x_half, cuts, cut_weights):
    return pl.pallas_call(
        _tc_body,
        grid=(N_TC_BLOCKS,),
        in_specs=[
            pl.BlockSpec((TC_BLOCK, INPUT_WIDTH), lambda i: (i, 0)),
            pl.BlockSpec((N_CUTS, INPUT_WIDTH), lambda i: (0, 0)),
            pl.BlockSpec((N_CUTS, 2, OUTPUT_WIDTH), lambda i: (0, 0, 0)),
        ],
        out_specs=[
            pl.BlockSpec((1, OUTPUT_WIDTH, TC_BLOCK), lambda i: (i, 0, 0)),
            pl.BlockSpec((1, 1, TC_BLOCK), lambda i: (i, 0, 0)),
        ],
        out_shape=[
            jax.ShapeDtypeStruct(
                (N_TC_BLOCKS, OUTPUT_WIDTH, TC_BLOCK), jnp.float32),
            jax.ShapeDtypeStruct((N_TC_BLOCKS, 1, TC_BLOCK), jnp.int32),
        ],
    )(x_half, cuts, cut_weights)


def _sc_body(tflat_hbm, idx3_hbm, cview_hbm, out4_hbm,
             idx_v, base_v, offs_v, gath_v, contrib_v,
             csem, gsem0, gsem1, gsem2, gsem3):
    gsems = (gsem0, gsem1, gsem2, gsem3)
    wid = lax.axis_index("s") * NUM_SC_CORES + lax.axis_index("c")
    j = wid // W_PER_BLOCK
    q = wid % W_PER_BLOCK
    pltpu.sync_copy(idx3_hbm.at[j, :, pl.ds(q * B_PER_W, B_PER_W)], idx_v)
    ccp = pltpu.async_copy(
        cview_hbm.at[j, :, pl.ds(N_CHUNKS * q, N_CHUNKS)], contrib_v, csem)

    for k in range(B_PER_W // 16):
        sl = pl.ds(k * 16, 16)
        v = idx_v[0, sl]
        base_v[sl] = ((v >> 7) << 10) + (v & 127)

    def make_fire(gsem):
        def fire(c, carry):
            cb = c >> 3
            s = c & 7
            cconst = (cb << 23) + (s << 7)
            for k in range(B_PER_W // 16):
                sl = pl.ds(k * 16, 16)
                offs_v[c, k // 8, pl.ds((k % 8) * 16, 16)] = (
                    base_v[sl] + cconst)
            for ch in range(N_CHUNKS):
                pltpu.async_copy(
                    tflat_hbm.at[offs_v.at[c, ch]],
                    gath_v.at[cb, ch, s], gsem)
            return carry
        return fire

    for g in range(N_SEM_GROUPS):
        lax.fori_loop(g * C_PER_GROUP, (g + 1) * C_PER_GROUP,
                      make_fire(gsems[g]), 0)

    ccp.wait()

    def add(c, carry):
        cb = c >> 3
        s = c & 7
        for k in range(B_PER_W // 16):
            sl = pl.ds((k % 8) * 16, 16)
            gath_v[cb, k >> 3, s, sl] = (gath_v[cb, k >> 3, s, sl]
                                         + contrib_v[cb, k >> 3, s, sl])
        return carry

    for g in range(N_SEM_GROUPS):
        cbs = pl.ds(2 * g, 2)
        pltpu.make_async_copy(
            out4_hbm.at[cbs, pl.ds(N_CHUNKS * wid, N_CHUNKS)], gath_v.at[cbs],
            gsems[g]).wait()
        lax.fori_loop(g * C_PER_GROUP, (g + 1) * C_PER_GROUP, add, 0)

    pltpu.sync_copy(gath_v, out4_hbm.at[:, pl.ds(N_CHUNKS * wid, N_CHUNKS)])


def _sc_stage(tflat, idx3, cview):
    mesh = plsc.VectorSubcoreMesh(
        core_axis_name="c", subcore_axis_name="s",
        num_cores=NUM_SC_CORES, num_subcores=NUM_SC_SUBCORES)
    call = pl.kernel(
        _sc_body,
        mesh=mesh,
        compiler_params=pltpu.CompilerParams(use_tc_tiling_on_sc=False),
        out_type=jax.ShapeDtypeStruct((8, HALF // 128, 8, 128), jnp.float32),
        scratch_types=[
            pltpu.VMEM((1, B_PER_W), jnp.int32),
            pltpu.VMEM((B_PER_W,), jnp.int32),
            pltpu.VMEM((OUTPUT_WIDTH, N_CHUNKS, 128), jnp.int32),
            pltpu.VMEM((8, N_CHUNKS, 8, 128), jnp.float32),
            pltpu.VMEM((8, N_CHUNKS, 8, 128), jnp.float32),
            pltpu.SemaphoreType.DMA,
            pltpu.SemaphoreType.DMA,
            pltpu.SemaphoreType.DMA,
            pltpu.SemaphoreType.DMA,
            pltpu.SemaphoreType.DMA,
        ],
    )
    return call(tflat, idx3, cview)


def _half(x_half, cuts, cut_weights, tflat):
    contrib3, idx3 = _tc_stage(x_half, cuts, cut_weights)
    cview = jnp.transpose(
        contrib3.reshape(N_TC_BLOCKS, 8, 8, 16, 128), (0, 1, 3, 2, 4))
    return _sc_stage(tflat, idx3, cview)


@jax.jit
def kernel(x, cuts, cut_weights, outputs_table):
    tflat = jnp.transpose(
        outputs_table.T.reshape(8, 8, 8192, 128), (0, 2, 1, 3)).reshape(-1)
    out4_a = _half(x[:HALF], cuts, cut_weights, tflat)
    out4_b = _half(x[HALF:], cuts, cut_weights, tflat)
    out4 = jnp.concatenate([out4_a, out4_b], axis=1)
    return jnp.transpose(out4, (1, 3, 0, 2)).reshape(BATCH, OUTPUT_WIDTH)


# speedup vs baseline: 1.1893x; 1.1893x over previous
import functools

import jax
import jax.numpy as jnp
from jax import lax
from jax.experimental import pallas as pl
from jax.experimental.pallas import tpu as pltpu
from jax.experimental.pallas import tpu_sc as plsc

INPUT_WIDTH = 128
OUTPUT_WIDTH = 64
N_CUTS = 20
BATCH = 16384

NUM_SC_CORES = 2
NUM_SC_SUBCORES = 16
NUM_WORKERS = NUM_SC_CORES * NUM_SC_SUBCORES

TC_BLOCK = 2048
N_TC_BLOCKS = BATCH // TC_BLOCK
B_PER_W = BATCH // NUM_WORKERS
N_CHUNKS = B_PER_W // 128
N_SEM_GROUPS = 4
C_PER_GROUP = OUTPUT_WIDTH // N_SEM_GROUPS


def _tc_body(x_ref, cuts_ref, cw_ref, contrib_ref, idx_ref):
    xb = x_ref[...]
    cuts = cuts_ref[...]
    logits_t = lax.dot_general(
        cuts, xb, (((1,), (1,)), ((), ())),
        preferred_element_type=jnp.float32)
    act = jax.nn.sigmoid(logits_t)
    w0 = cw_ref[:, 0, :]
    w1 = cw_ref[:, 1, :]
    wdiff = w0 - w1
    wbase = jnp.sum(w1, axis=0)
    contrib_t = lax.dot_general(
        wdiff, act, (((0,), (0,)), ((), ())),
        precision=lax.Precision.HIGHEST,
        preferred_element_type=jnp.float32) + wbase[:, None]
    contrib_ref[...] = contrib_t[None]
    pow2 = (2 ** (N_CUTS - 1 - lax.broadcasted_iota(
        jnp.int32, (N_CUTS, 1), 0)))
    bits = jnp.where(logits_t > 0.0, pow2, 0)
    idx_ref[...] = jnp.sum(bits, axis=0, keepdims=True)[None]


def _tc_stage(x, cuts, cut_weights):
    return pl.pallas_call(
        _tc_body,
        grid=(N_TC_BLOCKS,),
        in_specs=[
            pl.BlockSpec((TC_BLOCK, INPUT_WIDTH), lambda i: (i, 0)),
            pl.BlockSpec((N_CUTS, INPUT_WIDTH), lambda i: (0, 0)),
            pl.BlockSpec((N_CUTS, 2, OUTPUT_WIDTH), lambda i: (0, 0, 0)),
        ],
        out_specs=[
            pl.BlockSpec((1, OUTPUT_WIDTH, TC_BLOCK), lambda i: (i, 0, 0)),
            pl.BlockSpec((1, 1, TC_BLOCK), lambda i: (i, 0, 0)),
        ],
        out_shape=[
            jax.ShapeDtypeStruct(
                (N_TC_BLOCKS, OUTPUT_WIDTH, TC_BLOCK), jnp.float32),
            jax.ShapeDtypeStruct((N_TC_BLOCKS, 1, TC_BLOCK), jnp.int32),
        ],
    )(x, cuts, cut_weights)


def _sc_body(tflat_hbm, idx3_hbm, cview_hbm, out4_hbm,
             idx_v, base_v, offs_v, gath_v, contrib_v,
             csem, gsem0, gsem1, gsem2, gsem3):
    gsems = (gsem0, gsem1, gsem2, gsem3)
    wid = lax.axis_index("s") * NUM_SC_CORES + lax.axis_index("c")
    j = wid // 4
    q = wid % 4
    pltpu.sync_copy(idx3_hbm.at[j, :, pl.ds(q * B_PER_W, B_PER_W)], idx_v)
    ccp = pltpu.async_copy(
        cview_hbm.at[j, :, pl.ds(4 * q, 4)], contrib_v, csem)

    for k in range(B_PER_W // 16):
        sl = pl.ds(k * 16, 16)
        v = idx_v[0, sl]
        base_v[sl] = ((v >> 7) << 10) + (v & 127)

    def make_fire(gsem):
        def fire(c, carry):
            cb = c >> 3
            s = c & 7
            cconst = (cb << 23) + (s << 7)
            for k in range(B_PER_W // 16):
                sl = pl.ds(k * 16, 16)
                offs_v[c, k // 8, pl.ds((k % 8) * 16, 16)] = (
                    base_v[sl] + cconst)
            for ch in range(N_CHUNKS):
                pltpu.async_copy(
                    tflat_hbm.at[offs_v.at[c, ch]],
                    gath_v.at[cb, ch, s], gsem)
            return carry
        return fire

    for g in range(N_SEM_GROUPS):
        lax.fori_loop(g * C_PER_GROUP, (g + 1) * C_PER_GROUP,
                      make_fire(gsems[g]), 0)

    ccp.wait()

    def add(c, carry):
        cb = c >> 3
        s = c & 7
        for k in range(B_PER_W // 16):
            sl = pl.ds((k % 8) * 16, 16)
            gath_v[cb, k >> 3, s, sl] = (gath_v[cb, k >> 3, s, sl]
                                         + contrib_v[cb, k >> 3, s, sl])
        return carry

    for g in range(N_SEM_GROUPS):
        cbs = pl.ds(2 * g, 2)
        pltpu.make_async_copy(
            out4_hbm.at[cbs, pl.ds(4 * wid, 4)], gath_v.at[cbs],
            gsems[g]).wait()
        lax.fori_loop(g * C_PER_GROUP, (g + 1) * C_PER_GROUP, add, 0)

    pltpu.sync_copy(gath_v, out4_hbm.at[:, pl.ds(4 * wid, 4)])


def _sc_stage(tflat, idx3, cview):
    mesh = plsc.VectorSubcoreMesh(
        core_axis_name="c", subcore_axis_name="s",
        num_cores=NUM_SC_CORES, num_subcores=NUM_SC_SUBCORES)
    call = pl.kernel(
        _sc_body,
        mesh=mesh,
        compiler_params=pltpu.CompilerParams(use_tc_tiling_on_sc=False),
        out_type=jax.ShapeDtypeStruct((8, BATCH // 128, 8, 128), jnp.float32),
        scratch_types=[
            pltpu.VMEM((1, B_PER_W), jnp.int32),
            pltpu.VMEM((B_PER_W,), jnp.int32),
            pltpu.VMEM((OUTPUT_WIDTH, N_CHUNKS, 128), jnp.int32),
            pltpu.VMEM((8, N_CHUNKS, 8, 128), jnp.float32),
            pltpu.VMEM((8, 4, 8, 128), jnp.float32),
            pltpu.SemaphoreType.DMA,
            pltpu.SemaphoreType.DMA,
            pltpu.SemaphoreType.DMA,
            pltpu.SemaphoreType.DMA,
            pltpu.SemaphoreType.DMA,
        ],
    )
    return call(tflat, idx3, cview)


@jax.jit
def kernel(x, cuts, cut_weights, outputs_table):
    contrib3, idx3 = _tc_stage(x, cuts, cut_weights)
    tflat = jnp.transpose(
        outputs_table.T.reshape(8, 8, 8192, 128), (0, 2, 1, 3)).reshape(-1)
    cview = jnp.transpose(
        contrib3.reshape(N_TC_BLOCKS, 8, 8, 16, 128), (0, 1, 3, 2, 4))
    out4 = _sc_stage(tflat, idx3, cview)
    return jnp.transpose(out4, (1, 3, 0, 2)).reshape(BATCH, OUTPUT_WIDTH)


# speedup vs baseline: 1.2155x; 1.0220x over previous
import functools

import jax
import jax.numpy as jnp
from jax import lax
from jax.experimental import pallas as pl
from jax.experimental.pallas import tpu as pltpu
from jax.experimental.pallas import tpu_sc as plsc

INPUT_WIDTH = 128
OUTPUT_WIDTH = 64
N_CUTS = 20
BATCH = 16384

NUM_SC_CORES = 2
NUM_SC_SUBCORES = 16
NUM_WORKERS = NUM_SC_CORES * NUM_SC_SUBCORES

TC_BLOCK = 4096
N_TC_BLOCKS = BATCH // TC_BLOCK
W_PER_BLOCK = TC_BLOCK // 512
B_PER_W = BATCH // NUM_WORKERS
N_CHUNKS = B_PER_W // 128
N_SEM_GROUPS = 4
C_PER_GROUP = OUTPUT_WIDTH // N_SEM_GROUPS


def _tc_body(x_ref, cuts_ref, cw_ref, contrib_ref, idx_ref):
    xb = x_ref[...]
    cuts = cuts_ref[...]
    logits_t = lax.dot_general(
        cuts, xb, (((1,), (1,)), ((), ())),
        preferred_element_type=jnp.float32)
    act = jax.nn.sigmoid(logits_t)
    w0 = cw_ref[:, 0, :]
    w1 = cw_ref[:, 1, :]
    wdiff = w0 - w1
    wbase = jnp.sum(w1, axis=0)
    contrib_t = lax.dot_general(
        wdiff, act, (((0,), (0,)), ((), ())),
        precision=lax.Precision.HIGHEST,
        preferred_element_type=jnp.float32) + wbase[:, None]
    contrib_ref[...] = contrib_t[None]
    pow2 = (2 ** (N_CUTS - 1 - lax.broadcasted_iota(
        jnp.int32, (N_CUTS, 1), 0)))
    bits = jnp.where(logits_t > 0.0, pow2, 0)
    idx_ref[...] = jnp.sum(bits, axis=0, keepdims=True)[None]


def _tc_stage(x, cuts, cut_weights):
    return pl.pallas_call(
        _tc_body,
        grid=(N_TC_BLOCKS,),
        in_specs=[
            pl.BlockSpec((TC_BLOCK, INPUT_WIDTH), lambda i: (i, 0)),
            pl.BlockSpec((N_CUTS, INPUT_WIDTH), lambda i: (0, 0)),
            pl.BlockSpec((N_CUTS, 2, OUTPUT_WIDTH), lambda i: (0, 0, 0)),
        ],
        out_specs=[
            pl.BlockSpec((1, OUTPUT_WIDTH, TC_BLOCK), lambda i: (i, 0, 0)),
            pl.BlockSpec((1, 1, TC_BLOCK), lambda i: (i, 0, 0)),
        ],
        out_shape=[
            jax.ShapeDtypeStruct(
                (N_TC_BLOCKS, OUTPUT_WIDTH, TC_BLOCK), jnp.float32),
            jax.ShapeDtypeStruct((N_TC_BLOCKS, 1, TC_BLOCK), jnp.int32),
        ],
    )(x, cuts, cut_weights)


def _sc_body(tflat_hbm, idx3_hbm, cview_hbm, out4_hbm,
             idx_v, base_v, offs_v, gath_v, contrib_v,
             csem, gsem0, gsem1, gsem2, gsem3):
    gsems = (gsem0, gsem1, gsem2, gsem3)
    wid = lax.axis_index("s") * NUM_SC_CORES + lax.axis_index("c")
    j = wid // W_PER_BLOCK
    q = wid % W_PER_BLOCK
    pltpu.sync_copy(idx3_hbm.at[j, :, pl.ds(q * B_PER_W, B_PER_W)], idx_v)
    ccp = pltpu.async_copy(
        cview_hbm.at[j, :, pl.ds(4 * q, 4)], contrib_v, csem)

    for k in range(B_PER_W // 16):
        sl = pl.ds(k * 16, 16)
        v = idx_v[0, sl]
        base_v[sl] = ((v >> 7) << 10) + (v & 127)

    def make_fire(gsem):
        def fire(c, carry):
            cb = c >> 3
            s = c & 7
            cconst = (cb << 23) + (s << 7)
            for k in range(B_PER_W // 16):
                sl = pl.ds(k * 16, 16)
                offs_v[c, k // 8, pl.ds((k % 8) * 16, 16)] = (
                    base_v[sl] + cconst)
            for ch in range(N_CHUNKS):
                pltpu.async_copy(
                    tflat_hbm.at[offs_v.at[c, ch]],
                    gath_v.at[cb, ch, s], gsem)
            return carry
        return fire

    for g in range(N_SEM_GROUPS):
        lax.fori_loop(g * C_PER_GROUP, (g + 1) * C_PER_GROUP,
                      make_fire(gsems[g]), 0)

    ccp.wait()

    def add(c, carry):
        cb = c >> 3
        s = c & 7
        for k in range(B_PER_W // 16):
            sl = pl.ds((k % 8) * 16, 16)
            gath_v[cb, k >> 3, s, sl] = (gath_v[cb, k >> 3, s, sl]
                                         + contrib_v[cb, k >> 3, s, sl])
        return carry

    for g in range(N_SEM_GROUPS):
        cbs = pl.ds(2 * g, 2)
        pltpu.make_async_copy(
            out4_hbm.at[cbs, pl.ds(4 * wid, 4)], gath_v.at[cbs],
            gsems[g]).wait()
        lax.fori_loop(g * C_PER_GROUP, (g + 1) * C_PER_GROUP, add, 0)

    pltpu.sync_copy(gath_v, out4_hbm.at[:, pl.ds(4 * wid, 4)])


def _sc_stage(tflat, idx3, cview):
    mesh = plsc.VectorSubcoreMesh(
        core_axis_name="c", subcore_axis_name="s",
        num_cores=NUM_SC_CORES, num_subcores=NUM_SC_SUBCORES)
    call = pl.kernel(
        _sc_body,
        mesh=mesh,
        compiler_params=pltpu.CompilerParams(use_tc_tiling_on_sc=False),
        out_type=jax.ShapeDtypeStruct((8, BATCH // 128, 8, 128), jnp.float32),
        scratch_types=[
            pltpu.VMEM((1, B_PER_W), jnp.int32),
            pltpu.VMEM((B_PER_W,), jnp.int32),
            pltpu.VMEM((OUTPUT_WIDTH, N_CHUNKS, 128), jnp.int32),
            pltpu.VMEM((8, N_CHUNKS, 8, 128), jnp.float32),
            pltpu.VMEM((8, 4, 8, 128), jnp.float32),
            pltpu.SemaphoreType.DMA,
            pltpu.SemaphoreType.DMA,
            pltpu.SemaphoreType.DMA,
            pltpu.SemaphoreType.DMA,
            pltpu.SemaphoreType.DMA,
        ],
    )
    return call(tflat, idx3, cview)


@jax.jit
def kernel(x, cuts, cut_weights, outputs_table):
    contrib3, idx3 = _tc_stage(x, cuts, cut_weights)
    tflat = jnp.transpose(
        outputs_table.T.reshape(8, 8, 8192, 128), (0, 2, 1, 3)).reshape(-1)
    cview = jnp.transpose(
        contrib3.reshape(N_TC_BLOCKS, 8, 8, TC_BLOCK // 128, 128),
        (0, 1, 3, 2, 4))
    out4 = _sc_stage(tflat, idx3, cview)
    return jnp.transpose(out4, (1, 3, 0, 2)).reshape(BATCH, OUTPUT_WIDTH)


# speedup vs baseline: 1.2262x; 1.0088x over previous
import functools

import jax
import jax.numpy as jnp
from jax import lax
from jax.experimental import pallas as pl
from jax.experimental.pallas import tpu as pltpu
from jax.experimental.pallas import tpu_sc as plsc

INPUT_WIDTH = 128
OUTPUT_WIDTH = 64
N_CUTS = 20
BATCH = 16384

NUM_SC_CORES = 2
NUM_SC_SUBCORES = 16
NUM_WORKERS = NUM_SC_CORES * NUM_SC_SUBCORES

TC_BLOCK = 8192
N_TC_BLOCKS = BATCH // TC_BLOCK
W_PER_BLOCK = TC_BLOCK // 512
B_PER_W = BATCH // NUM_WORKERS
N_CHUNKS = B_PER_W // 128
N_SEM_GROUPS = 4
C_PER_GROUP = OUTPUT_WIDTH // N_SEM_GROUPS


def _tc_body(x_ref, cuts_ref, cw_ref, contrib_ref, idx_ref):
    xb = x_ref[...]
    cuts = cuts_ref[...]
    logits_t = lax.dot_general(
        cuts, xb, (((1,), (1,)), ((), ())),
        preferred_element_type=jnp.float32)
    act = jax.nn.sigmoid(logits_t)
    w0 = cw_ref[:, 0, :]
    w1 = cw_ref[:, 1, :]
    wdiff = w0 - w1
    wbase = jnp.sum(w1, axis=0)
    contrib_t = lax.dot_general(
        wdiff, act, (((0,), (0,)), ((), ())),
        precision=lax.Precision.HIGHEST,
        preferred_element_type=jnp.float32) + wbase[:, None]
    contrib_ref[...] = contrib_t[None]
    pow2 = (2 ** (N_CUTS - 1 - lax.broadcasted_iota(
        jnp.int32, (N_CUTS, 1), 0)))
    bits = jnp.where(logits_t > 0.0, pow2, 0)
    idx_ref[...] = jnp.sum(bits, axis=0, keepdims=True)[None]


def _tc_stage(x, cuts, cut_weights):
    return pl.pallas_call(
        _tc_body,
        grid=(N_TC_BLOCKS,),
        in_specs=[
            pl.BlockSpec((TC_BLOCK, INPUT_WIDTH), lambda i: (i, 0)),
            pl.BlockSpec((N_CUTS, INPUT_WIDTH), lambda i: (0, 0)),
            pl.BlockSpec((N_CUTS, 2, OUTPUT_WIDTH), lambda i: (0, 0, 0)),
        ],
        out_specs=[
            pl.BlockSpec((1, OUTPUT_WIDTH, TC_BLOCK), lambda i: (i, 0, 0)),
            pl.BlockSpec((1, 1, TC_BLOCK), lambda i: (i, 0, 0)),
        ],
        out_shape=[
            jax.ShapeDtypeStruct(
                (N_TC_BLOCKS, OUTPUT_WIDTH, TC_BLOCK), jnp.float32),
            jax.ShapeDtypeStruct((N_TC_BLOCKS, 1, TC_BLOCK), jnp.int32),
        ],
    )(x, cuts, cut_weights)


def _sc_body(tflat_hbm, idx3_hbm, cview_hbm, out4_hbm,
             idx_v, base_v, offs_v, gath_v, contrib_v,
             csem, gsem0, gsem1, gsem2, gsem3):
    gsems = (gsem0, gsem1, gsem2, gsem3)
    wid = lax.axis_index("s") * NUM_SC_CORES + lax.axis_index("c")
    j = wid // W_PER_BLOCK
    q = wid % W_PER_BLOCK
    pltpu.sync_copy(idx3_hbm.at[j, :, pl.ds(q * B_PER_W, B_PER_W)], idx_v)
    ccp = pltpu.async_copy(
        cview_hbm.at[j, :, pl.ds(4 * q, 4)], contrib_v, csem)

    for k in range(B_PER_W // 16):
        sl = pl.ds(k * 16, 16)
        v = idx_v[0, sl]
        base_v[sl] = ((v >> 7) << 10) + (v & 127)

    def make_fire(gsem):
        def fire(c, carry):
            cb = c >> 3
            s = c & 7
            cconst = (cb << 23) + (s << 7)
            for k in range(B_PER_W // 16):
                sl = pl.ds(k * 16, 16)
                offs_v[c, k // 8, pl.ds((k % 8) * 16, 16)] = (
                    base_v[sl] + cconst)
            for ch in range(N_CHUNKS):
                pltpu.async_copy(
                    tflat_hbm.at[offs_v.at[c, ch]],
                    gath_v.at[cb, ch, s], gsem)
            return carry
        return fire

    for g in range(N_SEM_GROUPS):
        lax.fori_loop(g * C_PER_GROUP, (g + 1) * C_PER_GROUP,
                      make_fire(gsems[g]), 0)

    ccp.wait()

    def add(c, carry):
        cb = c >> 3
        s = c & 7
        for k in range(B_PER_W // 16):
            sl = pl.ds((k % 8) * 16, 16)
            gath_v[cb, k >> 3, s, sl] = (gath_v[cb, k >> 3, s, sl]
                                         + contrib_v[cb, k >> 3, s, sl])
        return carry

    for g in range(N_SEM_GROUPS):
        cbs = pl.ds(2 * g, 2)
        pltpu.make_async_copy(
            out4_hbm.at[cbs, pl.ds(4 * wid, 4)], gath_v.at[cbs],
            gsems[g]).wait()
        lax.fori_loop(g * C_PER_GROUP, (g + 1) * C_PER_GROUP, add, 0)

    pltpu.sync_copy(gath_v, out4_hbm.at[:, pl.ds(4 * wid, 4)])


def _sc_stage(tflat, idx3, cview):
    mesh = plsc.VectorSubcoreMesh(
        core_axis_name="c", subcore_axis_name="s",
        num_cores=NUM_SC_CORES, num_subcores=NUM_SC_SUBCORES)
    call = pl.kernel(
        _sc_body,
        mesh=mesh,
        compiler_params=pltpu.CompilerParams(use_tc_tiling_on_sc=False),
        out_type=jax.ShapeDtypeStruct((8, BATCH // 128, 8, 128), jnp.float32),
        scratch_types=[
            pltpu.VMEM((1, B_PER_W), jnp.int32),
            pltpu.VMEM((B_PER_W,), jnp.int32),
            pltpu.VMEM((OUTPUT_WIDTH, N_CHUNKS, 128), jnp.int32),
            pltpu.VMEM((8, N_CHUNKS, 8, 128), jnp.float32),
            pltpu.VMEM((8, 4, 8, 128), jnp.float32),
            pltpu.SemaphoreType.DMA,
            pltpu.SemaphoreType.DMA,
            pltpu.SemaphoreType.DMA,
            pltpu.SemaphoreType.DMA,
            pltpu.SemaphoreType.DMA,
        ],
    )
    return call(tflat, idx3, cview)


@jax.jit
def kernel(x, cuts, cut_weights, outputs_table):
    contrib3, idx3 = _tc_stage(x, cuts, cut_weights)
    tflat = jnp.transpose(
        outputs_table.T.reshape(8, 8, 8192, 128), (0, 2, 1, 3)).reshape(-1)
    cview = jnp.transpose(
        contrib3.reshape(N_TC_BLOCKS, 8, 8, TC_BLOCK // 128, 128),
        (0, 1, 3, 2, 4))
    out4 = _sc_stage(tflat, idx3, cview)
    return jnp.transpose(out4, (1, 3, 0, 2)).reshape(BATCH, OUTPUT_WIDTH)
